# baseline (speedup 1.0000x reference)
import functools

import jax
import jax.numpy as jnp
from jax.experimental import pallas as pl
from jax.experimental.pallas import tpu as pltpu

N = 4096
D1, D2, D3, D_IN = 128, 256, 512, 512


def _l1_kernel(z_ref, w_ref, o_ref):
    acc = jnp.dot(z_ref[...], w_ref[...], preferred_element_type=jnp.float32)
    o_ref[...] = jnp.tanh(acc).astype(jnp.bfloat16)


def _layer1(z_bf, w4_bf, bm=1024):
    return pl.pallas_call(
        _l1_kernel,
        grid=(N // bm,),
        in_specs=[
            pl.BlockSpec((bm, D1), lambda m: (m, 0)),
            pl.BlockSpec((D1, D2), lambda m: (0, 0)),
        ],
        out_specs=pl.BlockSpec((bm, D2), lambda m: (m, 0)),
        out_shape=jax.ShapeDtypeStruct((N, D2), jnp.bfloat16),
        compiler_params=pltpu.CompilerParams(
            dimension_semantics=("parallel",),
        ),
    )(z_bf, w4_bf)


def _gnn_kernel(a_ref, s_ref, w_ref, o_ref, acc_ref, *, k_steps, act, out_dtype):
    @pl.when(pl.program_id(1) == 0)
    def _init():
        acc_ref[...] = jnp.zeros_like(acc_ref)

    acc_ref[...] += jnp.dot(
        a_ref[...], s_ref[...], preferred_element_type=jnp.float32
    )

    @pl.when(pl.program_id(1) == k_steps - 1)
    def _epilogue():
        r = jnp.dot(
            acc_ref[...].astype(jnp.bfloat16),
            w_ref[...],
            preferred_element_type=jnp.float32,
        )
        if act:
            r = jnp.tanh(r)
        o_ref[...] = r.astype(out_dtype)


def _gnn_layer(adj_bf, s_bf, w_bf, act, out_dtype, bm=512, bk=512):
    d_in = s_bf.shape[1]
    d_out = w_bf.shape[1]
    k_steps = N // bk
    kern = functools.partial(
        _gnn_kernel, k_steps=k_steps, act=act, out_dtype=out_dtype
    )
    return pl.pallas_call(
        kern,
        grid=(N // bm, k_steps),
        in_specs=[
            pl.BlockSpec((bm, bk), lambda m, k: (m, k)),
            pl.BlockSpec((bk, d_in), lambda m, k: (k, 0)),
            pl.BlockSpec((d_in, d_out), lambda m, k: (0, 0)),
        ],
        out_specs=pl.BlockSpec((bm, d_out), lambda m, k: (m, 0)),
        out_shape=jax.ShapeDtypeStruct((N, d_out), out_dtype),
        scratch_shapes=[pltpu.VMEM((bm, d_in), jnp.float32)],
        compiler_params=pltpu.CompilerParams(
            dimension_semantics=("parallel", "arbitrary"),
        ),
    )(adj_bf, s_bf, w_bf)


def _adjmm_kernel(a_ref, s_ref, o_ref, acc_ref, *, k_steps):
    @pl.when(pl.program_id(1) == 0)
    def _init():
        acc_ref[...] = jnp.zeros_like(acc_ref)

    acc_ref[...] += jnp.dot(
        a_ref[...], s_ref[...], preferred_element_type=jnp.float32
    )

    @pl.when(pl.program_id(1) == k_steps - 1)
    def _epilogue():
        o_ref[...] = acc_ref[...]


def _adj_mm(adj_bf, s_bf, bm=512, bk=512):
    d = s_bf.shape[1]
    k_steps = N // bk
    return pl.pallas_call(
        functools.partial(_adjmm_kernel, k_steps=k_steps),
        grid=(N // bm, k_steps),
        in_specs=[
            pl.BlockSpec((bm, bk), lambda m, k: (m, k)),
            pl.BlockSpec((bk, d), lambda m, k: (k, 0)),
        ],
        out_specs=pl.BlockSpec((bm, d), lambda m, k: (m, 0)),
        out_shape=jax.ShapeDtypeStruct((N, d), jnp.float32),
        scratch_shapes=[pltpu.VMEM((bm, d), jnp.float32)],
        compiler_params=pltpu.CompilerParams(
            dimension_semantics=("parallel", "arbitrary"),
        ),
    )(adj_bf, s_bf)


def _recon_kernel(a_ref, b_ref, o_ref):
    acc = jax.lax.dot_general(
        a_ref[...],
        b_ref[...],
        dimension_numbers=(((1,), (1,)), ((), ())),
        preferred_element_type=jnp.float32,
    )
    o_ref[...] = jax.nn.sigmoid(acc)


def _recon(zh_bf, bm=512, bn=512):
    return pl.pallas_call(
        _recon_kernel,
        grid=(N // bm, N // bn),
        in_specs=[
            pl.BlockSpec((bm, D_IN), lambda i, j: (i, 0)),
            pl.BlockSpec((bn, D_IN), lambda i, j: (j, 0)),
        ],
        out_specs=pl.BlockSpec((bm, bn), lambda i, j: (i, j)),
        out_shape=jax.ShapeDtypeStruct((N, N), jnp.float32),
        compiler_params=pltpu.CompilerParams(
            dimension_semantics=("parallel", "parallel"),
        ),
    )(zh_bf, zh_bf)


def kernel(z_igae, adj, W4, W5, W6):
    bf = jnp.bfloat16
    adj_bf = adj.astype(bf)
    s1 = _layer1(z_igae.astype(bf), W4.astype(bf))
    s2 = _gnn_layer(adj_bf, s1, W5.astype(bf), act=True, out_dtype=bf)
    s3 = _gnn_layer(adj_bf, s2, W6.astype(bf), act=False, out_dtype=bf)
    z_hat = _adj_mm(adj_bf, s3)
    z_hat_adj = _recon(z_hat.astype(bf))
    return (z_hat, z_hat_adj)


# speedup vs baseline: 1.9708x; 1.9708x over previous
import functools

import jax
import jax.numpy as jnp
from jax.experimental import pallas as pl
from jax.experimental.pallas import tpu as pltpu

N = 4096
D1, D2, D3, D_IN = 128, 256, 512, 512


def _l1_kernel(z_ref, w_ref, o_ref):
    acc = jnp.dot(z_ref[...], w_ref[...], preferred_element_type=jnp.float32)
    o_ref[...] = jnp.tanh(acc).astype(jnp.bfloat16)


def _layer1(z_bf, w4_bf, bm=1024):
    return pl.pallas_call(
        _l1_kernel,
        grid=(N // bm,),
        in_specs=[
            pl.BlockSpec((bm, D1), lambda m: (m, 0)),
            pl.BlockSpec((D1, D2), lambda m: (0, 0)),
        ],
        out_specs=pl.BlockSpec((bm, D2), lambda m: (m, 0)),
        out_shape=jax.ShapeDtypeStruct((N, D2), jnp.bfloat16),
        compiler_params=pltpu.CompilerParams(
            dimension_semantics=("parallel",),
        ),
    )(z_bf, w4_bf)


def _gnn_kernel(a_ref, s_ref, w_ref, o_ref, *, act):
    acc = jnp.dot(a_ref[...], s_ref[...], preferred_element_type=jnp.float32)
    r = jnp.dot(
        acc.astype(jnp.bfloat16), w_ref[...], preferred_element_type=jnp.float32
    )
    if act:
        r = jnp.tanh(r)
    o_ref[...] = r.astype(jnp.bfloat16)


def _gnn_layer(adj_bf, s_bf, w_bf, act, bm=512):
    d_in = s_bf.shape[1]
    d_out = w_bf.shape[1]
    return pl.pallas_call(
        functools.partial(_gnn_kernel, act=act),
        grid=(N // bm,),
        in_specs=[
            pl.BlockSpec((bm, N), lambda m: (m, 0)),
            pl.BlockSpec((N, d_in), lambda m: (0, 0)),
            pl.BlockSpec((d_in, d_out), lambda m: (0, 0)),
        ],
        out_specs=pl.BlockSpec((bm, d_out), lambda m: (m, 0)),
        out_shape=jax.ShapeDtypeStruct((N, d_out), jnp.bfloat16),
        compiler_params=pltpu.CompilerParams(
            dimension_semantics=("parallel",),
        ),
    )(adj_bf, s_bf, w_bf)


def _adjmm_kernel(a_ref, s_ref, o32_ref, o16_ref):
    acc = jnp.dot(a_ref[...], s_ref[...], preferred_element_type=jnp.float32)
    o32_ref[...] = acc
    o16_ref[...] = acc.astype(jnp.bfloat16)


def _adj_mm(adj_bf, s_bf, bm=512):
    d = s_bf.shape[1]
    return pl.pallas_call(
        _adjmm_kernel,
        grid=(N // bm,),
        in_specs=[
            pl.BlockSpec((bm, N), lambda m: (m, 0)),
            pl.BlockSpec((N, d), lambda m: (0, 0)),
        ],
        out_specs=[
            pl.BlockSpec((bm, d), lambda m: (m, 0)),
            pl.BlockSpec((bm, d), lambda m: (m, 0)),
        ],
        out_shape=[
            jax.ShapeDtypeStruct((N, d), jnp.float32),
            jax.ShapeDtypeStruct((N, d), jnp.bfloat16),
        ],
        compiler_params=pltpu.CompilerParams(
            dimension_semantics=("parallel",),
        ),
    )(adj_bf, s_bf)


def _recon_kernel(a_ref, b_ref, o_ref):
    acc = jax.lax.dot_general(
        a_ref[...],
        b_ref[...],
        dimension_numbers=(((1,), (1,)), ((), ())),
        preferred_element_type=jnp.float32,
    )
    o_ref[...] = 0.5 * (1.0 + jnp.tanh(0.5 * acc))


def _recon(zh_bf, bm=1024, bn=1024):
    return pl.pallas_call(
        _recon_kernel,
        grid=(N // bm, N // bn),
        in_specs=[
            pl.BlockSpec((bm, D_IN), lambda i, j: (i, 0)),
            pl.BlockSpec((bn, D_IN), lambda i, j: (j, 0)),
        ],
        out_specs=pl.BlockSpec((bm, bn), lambda i, j: (i, j)),
        out_shape=jax.ShapeDtypeStruct((N, N), jnp.float32),
        compiler_params=pltpu.CompilerParams(
            dimension_semantics=("parallel", "parallel"),
        ),
    )(zh_bf, zh_bf)


def kernel(z_igae, adj, W4, W5, W6):
    bf = jnp.bfloat16
    adj_bf = adj.astype(bf)
    s1 = _layer1(z_igae.astype(bf), W4.astype(bf))
    s2 = _gnn_layer(adj_bf, s1, W5.astype(bf), act=True)
    s3 = _gnn_layer(adj_bf, s2, W6.astype(bf), act=False)
    z_hat, zh_bf = _adj_mm(adj_bf, s3)
    z_hat_adj = _recon(zh_bf)
    return (z_hat, z_hat_adj)


# speedup vs baseline: 2.3728x; 1.2040x over previous
import functools

import jax
import jax.numpy as jnp
from jax.experimental import pallas as pl
from jax.experimental.pallas import tpu as pltpu

N = 4096
D1, D2, D3, D_IN = 128, 256, 512, 512


def _l1_kernel(z_ref, w_ref, o_ref):
    z = z_ref[...].astype(jnp.bfloat16)
    w = w_ref[...].astype(jnp.bfloat16)
    acc = jnp.dot(z, w, preferred_element_type=jnp.float32)
    o_ref[...] = jnp.tanh(acc).astype(jnp.bfloat16)


def _layer1(z, w4, bm=1024):
    return pl.pallas_call(
        _l1_kernel,
        grid=(N // bm,),
        in_specs=[
            pl.BlockSpec((bm, D1), lambda m: (m, 0)),
            pl.BlockSpec((D1, D2), lambda m: (0, 0)),
        ],
        out_specs=pl.BlockSpec((bm, D2), lambda m: (m, 0)),
        out_shape=jax.ShapeDtypeStruct((N, D2), jnp.bfloat16),
        compiler_params=pltpu.CompilerParams(
            dimension_semantics=("parallel",),
        ),
    )(z, w4)


def _l2_kernel(a_ref, s_ref, w_ref, o_ref, abf_ref):
    a_bf = a_ref[...].astype(jnp.bfloat16)
    abf_ref[...] = a_bf
    acc = jnp.dot(a_bf, s_ref[...], preferred_element_type=jnp.float32)
    w = w_ref[...].astype(jnp.bfloat16)
    r = jnp.dot(acc.astype(jnp.bfloat16), w, preferred_element_type=jnp.float32)
    o_ref[...] = jnp.tanh(r).astype(jnp.bfloat16)


def _layer2(adj, s1_bf, w5, bm=512):
    return pl.pallas_call(
        _l2_kernel,
        grid=(N // bm,),
        in_specs=[
            pl.BlockSpec((bm, N), lambda m: (m, 0)),
            pl.BlockSpec((N, D2), lambda m: (0, 0)),
            pl.BlockSpec((D2, D3), lambda m: (0, 0)),
        ],
        out_specs=[
            pl.BlockSpec((bm, D3), lambda m: (m, 0)),
            pl.BlockSpec((bm, N), lambda m: (m, 0)),
        ],
        out_shape=[
            jax.ShapeDtypeStruct((N, D3), jnp.bfloat16),
            jax.ShapeDtypeStruct((N, N), jnp.bfloat16),
        ],
        compiler_params=pltpu.CompilerParams(
            dimension_semantics=("parallel",),
        ),
    )(adj, s1_bf, w5)


def _l3_kernel(a_ref, s_ref, w_ref, o_ref):
    acc = jnp.dot(a_ref[...], s_ref[...], preferred_element_type=jnp.float32)
    w = w_ref[...].astype(jnp.bfloat16)
    r = jnp.dot(acc.astype(jnp.bfloat16), w, preferred_element_type=jnp.float32)
    o_ref[...] = r.astype(jnp.bfloat16)


def _layer3(adj_bf, s2_bf, w6, bm=512):
    return pl.pallas_call(
        _l3_kernel,
        grid=(N // bm,),
        in_specs=[
            pl.BlockSpec((bm, N), lambda m: (m, 0)),
            pl.BlockSpec((N, D3), lambda m: (0, 0)),
            pl.BlockSpec((D3, D_IN), lambda m: (0, 0)),
        ],
        out_specs=pl.BlockSpec((bm, D_IN), lambda m: (m, 0)),
        out_shape=jax.ShapeDtypeStruct((N, D_IN), jnp.bfloat16),
        compiler_params=pltpu.CompilerParams(
            dimension_semantics=("parallel",),
        ),
    )(adj_bf, s2_bf, w6)


def _adjmm_kernel(a_ref, s_ref, o32_ref, o16_ref):
    acc = jnp.dot(a_ref[...], s_ref[...], preferred_element_type=jnp.float32)
    o32_ref[...] = acc
    o16_ref[...] = acc.astype(jnp.bfloat16)


def _adj_mm(adj_bf, s_bf, bm=512):
    d = s_bf.shape[1]
    return pl.pallas_call(
        _adjmm_kernel,
        grid=(N // bm,),
        in_specs=[
            pl.BlockSpec((bm, N), lambda m: (m, 0)),
            pl.BlockSpec((N, d), lambda m: (0, 0)),
        ],
        out_specs=[
            pl.BlockSpec((bm, d), lambda m: (m, 0)),
            pl.BlockSpec((bm, d), lambda m: (m, 0)),
        ],
        out_shape=[
            jax.ShapeDtypeStruct((N, d), jnp.float32),
            jax.ShapeDtypeStruct((N, d), jnp.bfloat16),
        ],
        compiler_params=pltpu.CompilerParams(
            dimension_semantics=("parallel",),
        ),
    )(adj_bf, s_bf)


def _recon_kernel(a_ref, b_ref, o_ref):
    acc = jax.lax.dot_general(
        a_ref[...],
        b_ref[...],
        dimension_numbers=(((1,), (1,)), ((), ())),
        preferred_element_type=jnp.float32,
    )
    o_ref[...] = 0.5 * (1.0 + jnp.tanh(0.5 * acc))


def _recon(zh_bf, bm=1024, bn=1024):
    return pl.pallas_call(
        _recon_kernel,
        grid=(N // bm, N // bn),
        in_specs=[
            pl.BlockSpec((bm, D_IN), lambda i, j: (i, 0)),
            pl.BlockSpec((bn, D_IN), lambda i, j: (j, 0)),
        ],
        out_specs=pl.BlockSpec((bm, bn), lambda i, j: (i, j)),
        out_shape=jax.ShapeDtypeStruct((N, N), jnp.float32),
        compiler_params=pltpu.CompilerParams(
            dimension_semantics=("parallel", "parallel"),
        ),
    )(zh_bf, zh_bf)


def kernel(z_igae, adj, W4, W5, W6):
    s1 = _layer1(z_igae, W4)
    s2, adj_bf = _layer2(adj, s1, W5)
    s3 = _layer3(adj_bf, s2, W6)
    z_hat, zh_bf = _adj_mm(adj_bf, s3)
    z_hat_adj = _recon(zh_bf)
    return (z_hat, z_hat_adj)


# speedup vs baseline: 2.8888x; 1.2175x over previous
import jax
import jax.numpy as jnp
from jax.experimental import pallas as pl
from jax.experimental.pallas import tpu as pltpu

N = 4096
D1, D2, D3, D_IN = 128, 256, 512, 512
BM = 512
TM = 1024


def _mega_kernel(z_ref, adj_ref, w4_ref, w5_ref, w6_ref,
                 zhat_ref, recon_ref,
                 s1_ref, s2_ref, s3_ref, zh_ref, *, np_, bm, tm, tj):
    s = pl.program_id(0)

    @pl.when(s == 0)
    def _s1():
        z = z_ref[...].astype(jnp.bfloat16)
        w4 = w4_ref[...].astype(jnp.bfloat16)
        acc = jnp.dot(z, w4, preferred_element_type=jnp.float32)
        s1_ref[...] = jnp.tanh(acc).astype(jnp.bfloat16)

    @pl.when(s < np_)
    def _stage_a():
        a = adj_ref[...].astype(jnp.bfloat16)
        acc = jnp.dot(a, s1_ref[...], preferred_element_type=jnp.float32)
        w5 = w5_ref[...].astype(jnp.bfloat16)
        r = jnp.dot(acc.astype(jnp.bfloat16), w5,
                    preferred_element_type=jnp.float32)
        s2_ref[pl.ds(s * bm, bm), :] = jnp.tanh(r).astype(jnp.bfloat16)

    @pl.when((s >= np_) & (s < 2 * np_))
    def _stage_b():
        a = adj_ref[...].astype(jnp.bfloat16)
        acc = jnp.dot(a, s2_ref[...], preferred_element_type=jnp.float32)
        w6 = w6_ref[...].astype(jnp.bfloat16)
        r = jnp.dot(acc.astype(jnp.bfloat16), w6,
                    preferred_element_type=jnp.float32)
        s3_ref[pl.ds((s - np_) * bm, bm), :] = r.astype(jnp.bfloat16)

    @pl.when((s >= 2 * np_) & (s < 3 * np_))
    def _stage_c():
        a = adj_ref[...].astype(jnp.bfloat16)
        acc = jnp.dot(a, s3_ref[...], preferred_element_type=jnp.float32)
        zhat_ref[...] = acc
        zh_ref[pl.ds((s - 2 * np_) * bm, bm), :] = acc.astype(jnp.bfloat16)

    @pl.when(s >= 3 * np_)
    def _stage_d():
        t = s - 3 * np_
        i = t // tj
        j = t % tj
        a = zh_ref[pl.ds(i * tm, tm), :]
        b = zh_ref[pl.ds(j * tm, tm), :]
        acc = jax.lax.dot_general(
            a, b, dimension_numbers=(((1,), (1,)), ((), ())),
            preferred_element_type=jnp.float32)
        recon_ref[...] = 0.5 * (1.0 + jnp.tanh(0.5 * acc))


def kernel(z_igae, adj, W4, W5, W6):
    n = N
    bm, tm = BM, TM
    np_ = n // bm
    tj = n // tm
    steps = 3 * np_ + tj * tj

    def adj_map(s):
        return (jnp.minimum(s, 3 * np_ - 1) % np_, 0)

    def zhat_map(s):
        return (jnp.clip(s - 2 * np_, 0, np_ - 1), 0)

    def recon_map(s):
        t = jnp.maximum(s - 3 * np_, 0)
        return (t // tj, t % tj)

    import functools
    kern = functools.partial(_mega_kernel, np_=np_, bm=bm, tm=tm, tj=tj)
    z_hat, z_hat_adj = pl.pallas_call(
        kern,
        grid=(steps,),
        in_specs=[
            pl.BlockSpec((n, D1), lambda s: (0, 0)),
            pl.BlockSpec((bm, n), adj_map),
            pl.BlockSpec((D1, D2), lambda s: (0, 0)),
            pl.BlockSpec((D2, D3), lambda s: (0, 0)),
            pl.BlockSpec((D3, D_IN), lambda s: (0, 0)),
        ],
        out_specs=[
            pl.BlockSpec((bm, D_IN), zhat_map),
            pl.BlockSpec((tm, tm), recon_map),
        ],
        out_shape=[
            jax.ShapeDtypeStruct((n, D_IN), jnp.float32),
            jax.ShapeDtypeStruct((n, n), jnp.float32),
        ],
        scratch_shapes=[
            pltpu.VMEM((n, D2), jnp.bfloat16),
            pltpu.VMEM((n, D3), jnp.bfloat16),
            pltpu.VMEM((n, D_IN), jnp.bfloat16),
            pltpu.VMEM((n, D_IN), jnp.bfloat16),
        ],
        compiler_params=pltpu.CompilerParams(
            dimension_semantics=("arbitrary",),
        ),
    )(z_igae, adj, W4, W5, W6)
    return (z_hat, z_hat_adj)
